# Initial kernel scaffold; baseline (speedup 1.0000x reference)
import jax
import jax.numpy as jnp
from jax.experimental import pallas as pl


def kernel(x, W):
    raise NotImplementedError("write your pallas kernel here")


# speedup vs baseline: 1.6728x; 1.6728x over previous
import functools

import jax
import jax.numpy as jnp
from jax.experimental import pallas as pl

TOKENS = 16384
DIM = 2048
N_EXPERTS = 64
TOPK = 8
BLOCK_T = 1024


def _router_kernel(x_ref, w_ref, w_out_ref, idx_out_ref, probs_out_ref):
    x = x_ref[...]
    w = w_ref[...]
    logits = jax.lax.dot_general(
        x, w,
        dimension_numbers=(((1,), (1,)), ((), ())),
        preferred_element_type=jnp.float32,
    )
    m = jnp.max(logits, axis=-1, keepdims=True)
    e = jnp.exp(logits - m)
    probs = e / jnp.sum(e, axis=-1, keepdims=True)
    probs_out_ref[...] = probs

    lane = jax.lax.broadcasted_iota(jnp.int32, probs.shape, 1)
    p = probs
    vals = []
    idxs = []
    for _ in range(TOPK):
        mx = jnp.max(p, axis=-1, keepdims=True)
        is_max = p == mx
        idx = jnp.min(jnp.where(is_max, lane, N_EXPERTS), axis=-1, keepdims=True)
        vals.append(mx)
        idxs.append(idx)
        p = jnp.where(lane == idx, -1.0, p)
    w_out_ref[...] = jnp.concatenate(vals, axis=-1)
    idx_out_ref[...] = jnp.concatenate(idxs, axis=-1)


@functools.partial(jax.jit, static_argnames=())
def kernel(x, W):
    grid = (TOKENS // BLOCK_T,)
    out_shapes = (
        jax.ShapeDtypeStruct((TOKENS, TOPK), jnp.float32),
        jax.ShapeDtypeStruct((TOKENS, TOPK), jnp.int32),
        jax.ShapeDtypeStruct((TOKENS, N_EXPERTS), jnp.float32),
    )
    weights, indices, probs = pl.pallas_call(
        _router_kernel,
        grid=grid,
        in_specs=[
            pl.BlockSpec((BLOCK_T, DIM), lambda i: (i, 0)),
            pl.BlockSpec((N_EXPERTS, DIM), lambda i: (0, 0)),
        ],
        out_specs=(
            pl.BlockSpec((BLOCK_T, TOPK), lambda i: (i, 0)),
            pl.BlockSpec((BLOCK_T, TOPK), lambda i: (i, 0)),
            pl.BlockSpec((BLOCK_T, N_EXPERTS), lambda i: (i, 0)),
        ),
        out_shape=out_shapes,
    )(x, W)
    return (weights, indices, probs)


# speedup vs baseline: 1.9484x; 1.1648x over previous
import functools

import jax
import jax.numpy as jnp
from jax.experimental import pallas as pl

TOKENS = 16384
DIM = 2048
N_EXPERTS = 64
TOPK = 8
BLOCK_T = 1024


def _router_kernel(x_ref, w_ref, w_out_ref, idx_out_ref, probs_out_ref):
    x = x_ref[...]
    w = w_ref[...]
    logits = jax.lax.dot_general(
        x, w,
        dimension_numbers=(((1,), (1,)), ((), ())),
        preferred_element_type=jnp.float32,
    )
    m = jnp.max(logits, axis=-1, keepdims=True)
    e = jnp.exp(logits - m)
    probs = e * (1.0 / jnp.sum(e, axis=-1, keepdims=True))
    probs_out_ref[...] = probs

    lane = jax.lax.broadcasted_iota(jnp.int32, probs.shape, 1)
    ikey = jax.lax.bitcast_convert_type(probs, jnp.int32)
    skey = (ikey & ~63) | (63 - lane)
    keys = []
    for _ in range(TOPK):
        mx = jnp.max(skey, axis=-1, keepdims=True)
        keys.append(mx)
        skey = jnp.where(skey == mx, jnp.iinfo(jnp.int32).min, skey)
    topkeys = jnp.concatenate(keys, axis=-1)
    idx_out_ref[...] = 63 - (topkeys & 63)
    w_out_ref[...] = jax.lax.bitcast_convert_type(topkeys & ~63, jnp.float32)


@functools.partial(jax.jit, static_argnames=())
def kernel(x, W):
    grid = (TOKENS // BLOCK_T,)
    out_shapes = (
        jax.ShapeDtypeStruct((TOKENS, TOPK), jnp.float32),
        jax.ShapeDtypeStruct((TOKENS, TOPK), jnp.int32),
        jax.ShapeDtypeStruct((TOKENS, N_EXPERTS), jnp.float32),
    )
    weights, indices, probs = pl.pallas_call(
        _router_kernel,
        grid=grid,
        in_specs=[
            pl.BlockSpec((BLOCK_T, DIM), lambda i: (i, 0)),
            pl.BlockSpec((N_EXPERTS, DIM), lambda i: (0, 0)),
        ],
        out_specs=(
            pl.BlockSpec((BLOCK_T, TOPK), lambda i: (i, 0)),
            pl.BlockSpec((BLOCK_T, TOPK), lambda i: (i, 0)),
            pl.BlockSpec((BLOCK_T, N_EXPERTS), lambda i: (i, 0)),
        ),
        out_shape=out_shapes,
    )(x, W)
    return (weights, indices, probs)


# speedup vs baseline: 1.9492x; 1.0004x over previous
import functools

import jax
import jax.numpy as jnp
from jax.experimental import pallas as pl
from jax.experimental.pallas import tpu as pltpu

TOKENS = 16384
DIM = 2048
N_EXPERTS = 64
TOPK = 8
BLOCK_T = 1024


def _router_kernel(x_ref, w_ref, w_out_ref, idx_out_ref, probs_out_ref):
    x = x_ref[...]
    w = w_ref[...]
    logits = jax.lax.dot_general(
        x, w,
        dimension_numbers=(((1,), (1,)), ((), ())),
        preferred_element_type=jnp.float32,
    )
    m = jnp.max(logits, axis=-1, keepdims=True)
    e = jnp.exp(logits - m)
    probs = e * (1.0 / jnp.sum(e, axis=-1, keepdims=True))
    probs_out_ref[...] = probs

    lane = jax.lax.broadcasted_iota(jnp.int32, probs.shape, 1)
    ikey = jax.lax.bitcast_convert_type(probs, jnp.int32)
    skey = (ikey & ~63) | (63 - lane)
    keys = []
    for _ in range(TOPK):
        mx = jnp.max(skey, axis=-1, keepdims=True)
        keys.append(mx)
        skey = jnp.where(skey == mx, jnp.iinfo(jnp.int32).min, skey)
    topkeys = jnp.concatenate(keys, axis=-1)
    idx_out_ref[...] = 63 - (topkeys & 63)
    w_out_ref[...] = jax.lax.bitcast_convert_type(topkeys & ~63, jnp.float32)


@functools.partial(jax.jit, static_argnames=())
def kernel(x, W):
    grid = (TOKENS // BLOCK_T,)
    out_shapes = (
        jax.ShapeDtypeStruct((TOKENS, TOPK), jnp.float32),
        jax.ShapeDtypeStruct((TOKENS, TOPK), jnp.int32),
        jax.ShapeDtypeStruct((TOKENS, N_EXPERTS), jnp.float32),
    )
    weights, indices, probs = pl.pallas_call(
        _router_kernel,
        grid=grid,
        in_specs=[
            pl.BlockSpec((BLOCK_T, DIM), lambda i: (i, 0)),
            pl.BlockSpec((N_EXPERTS, DIM), lambda i: (0, 0)),
        ],
        out_specs=(
            pl.BlockSpec((BLOCK_T, TOPK), lambda i: (i, 0)),
            pl.BlockSpec((BLOCK_T, TOPK), lambda i: (i, 0)),
            pl.BlockSpec((BLOCK_T, N_EXPERTS), lambda i: (i, 0)),
        ),
        out_shape=out_shapes,
        compiler_params=pltpu.CompilerParams(
            dimension_semantics=("parallel",),
        ),
    )(x, W)
    return (weights, indices, probs)
